# baseline (device time: 83037 ns/iter reference)
import jax
import jax.numpy as jnp
from jax import lax
from jax.experimental import pallas as pl
from jax.experimental.pallas import tpu as pltpu

N_DEV = 32
LOG2 = 5
B, Sq, Hq, Dh = 2, 256, 4, 64
HD = Hq * Dh
ACC_L = HD + 128


def kernel(x, Wq, K_ext, V_ext, Wo):
    skv_loc = K_ext.shape[1]
    d_model = Wo.shape[1]
    K2 = K_ext.reshape(B, skv_loc, HD)
    V2 = V_ext.reshape(B, skv_loc, HD)

    def body(x_ref, wq_ref, k_ref, v_ref, wo_ref, out_ref,
             acc_ref, recv_ref, send_sems, recv_sems):
        my = lax.axis_index("i")

        bsem = pltpu.get_barrier_semaphore()
        for k in range(LOG2):
            pl.semaphore_signal(
                bsem, inc=1,
                device_id=(my ^ (1 << k),),
                device_id_type=pl.DeviceIdType.MESH,
            )
        pl.semaphore_wait(bsem, LOG2)

        qb = lax.broadcasted_iota(jnp.int32, (Sq, skv_loc), 0) // 64
        kb = lax.broadcasted_iota(jnp.int32, (Sq, skv_loc), 1) // 64 + 4 * my
        mask = (qb == kb) | (kb == 0) | ((qb + kb) % 3 == 0)

        lane128 = lax.broadcasted_iota(jnp.int32, (skv_loc, 128), 1)
        for b in range(B):
            q_b = jnp.dot(x_ref[b], wq_ref[...],
                          preferred_element_type=jnp.float32)
            l_blk = jnp.zeros((Sq, 128), jnp.float32)
            for h in range(Hq):
                q_bh = q_b[:, h * Dh:(h + 1) * Dh]
                k_bh = k_ref[b, :, h * Dh:(h + 1) * Dh]
                v_bh = v_ref[b, :, h * Dh:(h + 1) * Dh]
                s = lax.dot_general(
                    q_bh, k_bh, (((1,), (1,)), ((), ())),
                    preferred_element_type=jnp.float32) * 0.125
                w = jnp.where(mask, jnp.exp(s), 0.0)
                acc_ref[b, :, h * Dh:(h + 1) * Dh] = jnp.dot(
                    w, v_bh, preferred_element_type=jnp.float32)
                e_h = (lane128 == h).astype(jnp.float32)
                l_blk = l_blk + jnp.dot(
                    w, e_h, preferred_element_type=jnp.float32)
            acc_ref[b, :, HD:ACC_L] = l_blk

        for k in range(LOG2):
            partner = my ^ (1 << k)
            rdma = pltpu.make_async_remote_copy(
                src_ref=acc_ref,
                dst_ref=recv_ref.at[k],
                send_sem=send_sems.at[k],
                recv_sem=recv_sems.at[k],
                device_id=(partner,),
                device_id_type=pl.DeviceIdType.MESH,
            )
            rdma.start()
            rdma.wait()
            acc_ref[...] = acc_ref[...] + recv_ref[k]

        sub128 = lax.broadcasted_iota(jnp.int32, (128, Dh), 0)
        for b in range(B):
            l_all = acc_ref[b, :, HD:ACC_L]
            out_b = jnp.zeros((Sq, d_model), jnp.float32)
            for h in range(Hq):
                f_h = (sub128 == h).astype(jnp.float32)
                l_h = jnp.dot(l_all, f_h,
                              preferred_element_type=jnp.float32)
                ctx = acc_ref[b, :, h * Dh:(h + 1) * Dh] / l_h
                out_b = out_b + jnp.dot(
                    ctx, wo_ref[h * Dh:(h + 1) * Dh, :],
                    preferred_element_type=jnp.float32)
            out_ref[b, :, :] = out_b

    return pl.pallas_call(
        body,
        out_shape=jax.ShapeDtypeStruct((B, Sq, d_model), jnp.float32),
        in_specs=[pl.BlockSpec(memory_space=pltpu.VMEM)] * 5,
        out_specs=pl.BlockSpec(memory_space=pltpu.VMEM),
        scratch_shapes=[
            pltpu.VMEM((B, Sq, ACC_L), jnp.float32),
            pltpu.VMEM((LOG2, B, Sq, ACC_L), jnp.float32),
            pltpu.SemaphoreType.DMA((LOG2,)),
            pltpu.SemaphoreType.DMA((LOG2,)),
        ],
        compiler_params=pltpu.CompilerParams(collective_id=0),
    )(x, Wq, K2, V2, Wo)


# device time: 45065 ns/iter; 1.8426x vs baseline; 1.8426x over previous
import jax
import jax.numpy as jnp
from jax import lax
from jax.experimental import pallas as pl
from jax.experimental.pallas import tpu as pltpu

N_DEV = 32
LOG2 = 5
B, Sq, Hq, Dh = 2, 256, 4, 64
HD = Hq * Dh
ACC_R = Sq + 8


def kernel(x, Wq, K_ext, V_ext, Wo):
    skv_loc = K_ext.shape[1]
    d_model = Wo.shape[1]
    K2 = K_ext.reshape(B, skv_loc, HD)
    V2 = V_ext.reshape(B, skv_loc, HD)

    def body(x_ref, wq_ref, k_ref, v_ref, wo_ref, out_ref,
             acc_ref, send_ref, recv_ref, send_sems, recv_sems):
        my = lax.axis_index("i")

        bsem = pltpu.get_barrier_semaphore()
        for k in range(LOG2):
            pl.semaphore_signal(
                bsem, inc=1,
                device_id=(my ^ (1 << k),),
                device_id_type=pl.DeviceIdType.MESH,
            )
        pl.semaphore_wait(bsem, LOG2)

        qb = lax.broadcasted_iota(jnp.int32, (Sq, skv_loc), 0) // 64
        kb = lax.broadcasted_iota(jnp.int32, (Sq, skv_loc), 1) // 64 + 4 * my
        mask = (qb == kb) | (kb == 0) | ((qb + kb) % 3 == 0)

        lane8 = lax.broadcasted_iota(jnp.int32, (skv_loc, 8), 1)
        ident = (lax.broadcasted_iota(jnp.int32, (Sq, Sq), 0) ==
                 lax.broadcasted_iota(jnp.int32, (Sq, Sq), 1)
                 ).astype(jnp.float32)
        for b in range(B):
            q_b = jnp.dot(x_ref[b], wq_ref[...],
                          preferred_element_type=jnp.float32)
            l_blk = jnp.zeros((Sq, 8), jnp.float32)
            for h in range(Hq):
                q_bh = q_b[:, h * Dh:(h + 1) * Dh]
                k_bh = k_ref[b, :, h * Dh:(h + 1) * Dh]
                v_bh = v_ref[b, :, h * Dh:(h + 1) * Dh]
                s = lax.dot_general(
                    q_bh, k_bh, (((1,), (1,)), ((), ())),
                    preferred_element_type=jnp.float32) * 0.125
                w = jnp.where(mask, jnp.exp(s), 0.0)
                acc_ref[b, :Sq, h * Dh:(h + 1) * Dh] = jnp.dot(
                    w, v_bh, preferred_element_type=jnp.float32)
                e_h = (lane8 == h).astype(jnp.float32)
                l_blk = l_blk + jnp.dot(
                    w, e_h, preferred_element_type=jnp.float32)
            l_rows = lax.dot_general(
                l_blk, ident, (((0,), (0,)), ((), ())),
                preferred_element_type=jnp.float32)
            acc_ref[b, Sq:ACC_R, :] = l_rows

        for k in range(LOG2):
            partner = my ^ (1 << k)
            send_ref[...] = acc_ref[...].astype(jnp.bfloat16)
            rdma = pltpu.make_async_remote_copy(
                src_ref=send_ref,
                dst_ref=recv_ref.at[k],
                send_sem=send_sems.at[k],
                recv_sem=recv_sems.at[k],
                device_id=(partner,),
                device_id_type=pl.DeviceIdType.MESH,
            )
            rdma.start()
            rdma.wait()
            acc_ref[...] = acc_ref[...] + recv_ref[k].astype(jnp.float32)

        sub8 = lax.broadcasted_iota(jnp.int32, (8, Dh), 0)
        for b in range(B):
            l_rows = acc_ref[b, Sq:ACC_R, :]
            out_b = jnp.zeros((Sq, d_model), jnp.float32)
            for h in range(Hq):
                g_h = (sub8 == h).astype(jnp.float32)
                l_h = lax.dot_general(
                    l_rows, g_h, (((0,), (0,)), ((), ())),
                    preferred_element_type=jnp.float32)
                ctx = acc_ref[b, :Sq, h * Dh:(h + 1) * Dh] / l_h
                out_b = out_b + jnp.dot(
                    ctx, wo_ref[h * Dh:(h + 1) * Dh, :],
                    preferred_element_type=jnp.float32)
            out_ref[b, :, :] = out_b

    return pl.pallas_call(
        body,
        out_shape=jax.ShapeDtypeStruct((B, Sq, d_model), jnp.float32),
        in_specs=[pl.BlockSpec(memory_space=pltpu.VMEM)] * 5,
        out_specs=pl.BlockSpec(memory_space=pltpu.VMEM),
        scratch_shapes=[
            pltpu.VMEM((B, ACC_R, HD), jnp.float32),
            pltpu.VMEM((B, ACC_R, HD), jnp.bfloat16),
            pltpu.VMEM((LOG2, B, ACC_R, HD), jnp.bfloat16),
            pltpu.SemaphoreType.DMA((LOG2,)),
            pltpu.SemaphoreType.DMA((LOG2,)),
        ],
        compiler_params=pltpu.CompilerParams(collective_id=0),
    )(x, Wq, K2, V2, Wo)


# device time: 38286 ns/iter; 2.1689x vs baseline; 1.1771x over previous
import jax
import jax.numpy as jnp
from jax import lax
from jax.experimental import pallas as pl
from jax.experimental.pallas import tpu as pltpu

N_DEV = 32
LOG2 = 5
B, Sq, Hq, Dh = 2, 256, 4, 64
HD = Hq * Dh
ACC_R = Sq + 8


def kernel(x, Wq, K_ext, V_ext, Wo):
    skv_loc = K_ext.shape[1]
    d_model = Wo.shape[1]
    K2 = K_ext.reshape(B, skv_loc, HD)
    V2 = V_ext.reshape(B, skv_loc, HD)

    def body(x_ref, wq_ref, k_ref, v_ref, wo_ref, out_ref,
             acc_ref, recv_ref, send_sems, recv_sems):
        my = lax.axis_index("i")

        bsem = pltpu.get_barrier_semaphore()
        for k in range(LOG2):
            pl.semaphore_signal(
                bsem, inc=1,
                device_id=(my ^ (1 << k),),
                device_id_type=pl.DeviceIdType.MESH,
            )
        pl.semaphore_wait(bsem, LOG2)

        qb = lax.broadcasted_iota(jnp.int32, (Sq, skv_loc), 0) // 64
        kb = lax.broadcasted_iota(jnp.int32, (Sq, skv_loc), 1) // 64 + 4 * my
        mask = (qb == kb) | (kb == 0) | ((qb + kb) % 3 == 0)

        lane8 = lax.broadcasted_iota(jnp.int32, (skv_loc, 8), 1)
        ident = (lax.broadcasted_iota(jnp.int32, (Sq, Sq), 0) ==
                 lax.broadcasted_iota(jnp.int32, (Sq, Sq), 1)
                 ).astype(jnp.float32)
        for b in range(B):
            q_b = jnp.dot(x_ref[b], wq_ref[...],
                          preferred_element_type=jnp.float32)
            l_blk = jnp.zeros((Sq, 8), jnp.float32)
            for h in range(Hq):
                q_bh = q_b[:, h * Dh:(h + 1) * Dh]
                k_bh = k_ref[b, :, h * Dh:(h + 1) * Dh]
                v_bh = v_ref[b, :, h * Dh:(h + 1) * Dh]
                s = lax.dot_general(
                    q_bh, k_bh, (((1,), (1,)), ((), ())),
                    preferred_element_type=jnp.float32) * 0.125
                w = jnp.where(mask, jnp.exp(s), 0.0)
                acc_ref[b, :Sq, h * Dh:(h + 1) * Dh] = jnp.dot(
                    w, v_bh, preferred_element_type=jnp.float32
                ).astype(jnp.bfloat16)
                e_h = (lane8 == h).astype(jnp.float32)
                l_blk = l_blk + jnp.dot(
                    w, e_h, preferred_element_type=jnp.float32)
            l_rows = lax.dot_general(
                l_blk, ident, (((0,), (0,)), ((), ())),
                preferred_element_type=jnp.float32)
            acc_ref[b, Sq:ACC_R, :] = l_rows.astype(jnp.bfloat16)

        def mk_rdma(k, b):
            return pltpu.make_async_remote_copy(
                src_ref=acc_ref.at[b],
                dst_ref=recv_ref.at[k, b],
                send_sem=send_sems.at[k, b],
                recv_sem=recv_sems.at[k, b],
                device_id=(my ^ (1 << k),),
                device_id_type=pl.DeviceIdType.MESH,
            )

        for b in range(B):
            mk_rdma(0, b).start()
        for k in range(LOG2):
            for b in range(B):
                rdma = mk_rdma(k, b)
                rdma.wait()
                acc_ref[b, :, :] = acc_ref[b, :, :] + recv_ref[k, b, :, :]
                if k + 1 < LOG2:
                    mk_rdma(k + 1, b).start()

        sub8 = lax.broadcasted_iota(jnp.int32, (8, Dh), 0)
        for b in range(B):
            l_rows = acc_ref[b, Sq:ACC_R, :].astype(jnp.float32)
            out_b = jnp.zeros((Sq, d_model), jnp.float32)
            for h in range(Hq):
                g_h = (sub8 == h).astype(jnp.float32)
                l_h = lax.dot_general(
                    l_rows, g_h, (((0,), (0,)), ((), ())),
                    preferred_element_type=jnp.float32)
                ctx = acc_ref[b, :Sq, h * Dh:(h + 1) * Dh].astype(
                    jnp.float32) / l_h
                out_b = out_b + jnp.dot(
                    ctx, wo_ref[h * Dh:(h + 1) * Dh, :],
                    preferred_element_type=jnp.float32)
            out_ref[b, :, :] = out_b

    return pl.pallas_call(
        body,
        out_shape=jax.ShapeDtypeStruct((B, Sq, d_model), jnp.float32),
        in_specs=[pl.BlockSpec(memory_space=pltpu.VMEM)] * 5,
        out_specs=pl.BlockSpec(memory_space=pltpu.VMEM),
        scratch_shapes=[
            pltpu.VMEM((B, ACC_R, HD), jnp.bfloat16),
            pltpu.VMEM((LOG2, B, ACC_R, HD), jnp.bfloat16),
            pltpu.SemaphoreType.DMA((LOG2, B)),
            pltpu.SemaphoreType.DMA((LOG2, B)),
        ],
        compiler_params=pltpu.CompilerParams(collective_id=0),
    )(x, Wq, K2, V2, Wo)
